# baseline (speedup 1.0000x reference)
import functools

import jax
import jax.numpy as jnp
from jax import lax
from jax.experimental import pallas as pl
from jax.experimental.pallas import tpu as pltpu
from jax.experimental.pallas import tpu_sc as plsc

B = 4096
L = 200
E = 64
AA_V = 22
POS_V = 210
NC = 2
NS = 16
NW = NC * NS
RPW = B // NW
NBUF = 4
PR = L * E // 128


def _embed_body(seqs_hbm, lens_hbm, aa_hbm, pos_hbm, pospair_hbm, out_hbm,
                aa_v, pos_v, seq_v, len_v, out_v, posfull_sh, osem, psem):
    c = lax.axis_index("c")
    s = lax.axis_index("s")
    wid = s * NC + c
    base = wid * RPW

    pltpu.sync_copy(aa_hbm, aa_v)
    pltpu.sync_copy(pos_hbm, pos_v)
    pltpu.sync_copy(seqs_hbm.at[pl.ds(base * L, RPW * L)], seq_v)
    pltpu.sync_copy(lens_hbm.at[pl.ds(base, RPW)], len_v.at[pl.ds(0, RPW)])

    @pl.when(s == 0)
    def _():
        pltpu.sync_copy(pospair_hbm, posfull_sh)
    plsc.subcore_barrier()

    for i in range(NBUF):
        pltpu.async_copy(posfull_sh, out_v.at[i, pl.ds(0, PR)], psem.at[i])

    iota = lax.iota(jnp.int32, 16)
    cols = [iota + 16 * k for k in range(4)]

    def row_body(r, carry):
        row = base + r
        slot = lax.rem(r, NBUF)
        ln = len_v[pl.ds(r, 16)][0]
        t0 = r * L
        pltpu.make_async_copy(posfull_sh, out_v.at[slot, pl.ds(0, PR)],
                              psem.at[slot]).wait()

        def aa_add(t, j, colbase):
            s_b = plsc.load_gather(seq_v, [jnp.full((16,), t0 + j, jnp.int32)])
            s64 = s_b << 6
            for k in range(4):
                plsc.addupdate(out_v.at[slot, t, pl.ds(colbase + 16 * k, 16)],
                               plsc.load_gather(aa_v, [s64 + cols[k]]))

        def aa_only(t, j, colbase):
            s_b = plsc.load_gather(seq_v, [jnp.full((16,), t0 + j, jnp.int32)])
            s64 = s_b << 6
            for k in range(4):
                out_v[slot, t, pl.ds(colbase + 16 * k, 16)] = (
                    plsc.load_gather(aa_v, [s64 + cols[k]]))

        half1 = ln >> 1

        @plsc.parallel_loop(0, half1, 1, unroll=2)
        def pair_body(t):
            j = t * 2
            aa_add(t, j, 0)
            aa_add(t, j + 1, 64)

        @pl.when((ln & 1) == 1)
        def _():
            aa_add(half1, ln - 1, 0)
            aa_only(half1, ln, 64)

        @plsc.parallel_loop((ln + 1) >> 1, L // 2, 1, unroll=2)
        def pair_body2(t):
            j = t * 2
            aa_only(t, j, 0)
            aa_only(t, j + 1, 64)

        pltpu.async_copy(out_v.at[slot, pl.ds(0, PR)], out_hbm.at[row], osem)

        s2 = lax.rem(r + 2, NBUF)

        @pl.when(r + 2 < RPW)
        def _():
            @pl.when(r >= 2)
            def _():
                pltpu.make_async_copy(
                    out_v.at[s2, pl.ds(0, PR)],
                    out_hbm.at[row], osem).wait()
            pltpu.async_copy(posfull_sh, out_v.at[s2, pl.ds(0, PR)],
                             psem.at[s2])
        return carry

    lax.fori_loop(0, RPW, row_body, 0)
    for _ in range(4):
        pltpu.make_async_copy(out_v.at[0, pl.ds(0, PR)],
                              out_hbm.at[base], osem).wait()


@functools.partial(
    pl.kernel,
    out_type=jax.ShapeDtypeStruct((B, PR, 128), jnp.float32),
    mesh=plsc.VectorSubcoreMesh(core_axis_name="c", subcore_axis_name="s"),
    scratch_types=[
        pltpu.VMEM((AA_V * E,), jnp.float32),
        pltpu.VMEM((POS_V * E,), jnp.float32),
        pltpu.VMEM((RPW * L,), jnp.int32),
        pltpu.VMEM((RPW + 16,), jnp.int32),
        pltpu.VMEM((NBUF, 104, 128), jnp.float32),
        pltpu.VMEM_SHARED((PR, 128), jnp.float32),
        pltpu.SemaphoreType.DMA,
        pltpu.SemaphoreType.DMA((NBUF,)),
    ],
    compiler_params=pltpu.CompilerParams(
        needs_layout_passes=False, disable_bounds_checks=True),
)
def _embed(seqs_hbm, lens_hbm, aa_hbm, pos_hbm, pospair_hbm, out_hbm,
           aa_v, pos_v, seq_v, len_v, out_v, posfull_sh, osem, psem):
    _embed_body(seqs_hbm, lens_hbm, aa_hbm, pos_hbm, pospair_hbm, out_hbm,
                aa_v, pos_v, seq_v, len_v, out_v, posfull_sh, osem, psem)


def kernel(seqs, lens, aa_table, pos_table):
    pos_pair = pos_table[1:L + 1].reshape(PR, 128)
    out = _embed(seqs.reshape(B * L), lens,
                 aa_table.reshape(AA_V * E), pos_table.reshape(POS_V * E),
                 pos_pair)
    return out.reshape(B, L, E)
